# baseline (speedup 1.0000x reference)
import jax
import jax.numpy as jnp
from jax import lax
from jax.experimental import pallas as pl
from jax.experimental.pallas import tpu as pltpu
from jax.experimental.pallas import tpu_sc as plsc

_N = 16384
_C = 1000
_K = int(_N * 0.2)
_GAMMA = 2

_NTC = 12288
_NSC = _N - _NTC

_NC, _NS, _L = 2, 16, 16
_NW = _NC * _NS
_RPT = _NSC // _NW
_CH = 32
_NCHUNK = _RPT // _CH
_NFULL = 62
_TAIL = _C - _L
_NEW = _C - _NFULL * _L
_IMIN = -2**31
_IMAXP = 0x7FFFFFFF

_BT = 1024
_NBLK = _NTC // _BT

_GDN = lax.GatherDimensionNumbers(
    offset_dims=(), collapsed_slice_dims=(0,), start_index_map=(0,))


def _lgather(v, idx):
    return lax.gather(v, idx[:, None], _GDN, (1,),
                      mode=lax.GatherScatterMode.PROMISE_IN_BOUNDS)


def _sc_rows(x_hbm, t_hbm, a_hbm, m_hbm, s_hbm, tv_hbm, av_hbm,
             xbuf0, xbuf1, tbuf, abuf, mbuf, sbuf, tvbuf, avbuf,
             sem0, sem1):
    wid = lax.axis_index("s") * _NC + lax.axis_index("c")
    out0 = wid * _RPT
    row0 = _NTC + out0
    pltpu.sync_copy(t_hbm.at[pl.ds(row0, _RPT)], tbuf)
    pltpu.sync_copy(a_hbm, abuf)
    lane = lax.broadcasted_iota(jnp.int32, (_L,), 0)
    tailmask = lane >= (_L - _NEW)

    def hmax(v):
        for sh in (8, 4, 2, 1):
            v = jnp.maximum(v, _lgather(v, lane ^ sh))
        return v

    def hsum(v):
        for sh in (8, 4, 2, 1):
            v = v + _lgather(v, lane ^ sh)
        return v

    def start(ch, buf, sem):
        pltpu.async_copy(x_hbm.at[pl.ds(row0 + ch * _CH, _CH)], buf, sem)

    def wait(buf, sem):
        pltpu.make_async_copy(x_hbm.at[pl.ds(0, _CH)], buf, sem).wait()

    offsets = [c * _L for c in range(_NFULL)] + [_TAIL]

    def process(ch, buf):
        base = ch * _CH

        def group_body(b, _):
            tvec16 = tbuf[pl.ds(base + b * _L, _L)]

            def row_body(j2, carry):
                accm, accs, acctv = carry
                j = b * _L + j2
                tsp = _lgather(tvec16, jnp.full((_L,), j2, jnp.int32))
                ninf = jnp.full((_L,), -jnp.inf, jnp.float32)
                zv = jnp.zeros((_L,), jnp.float32)
                ms = [ninf] * 8
                tvs = [zv] * 4
                for k, off in enumerate(offsets):
                    v = buf[j, pl.ds(off, _L)]
                    ms[k % 8] = jnp.maximum(ms[k % 8], v)
                    tvs[k % 4] = jnp.where((lane + off) == tsp, v, tvs[k % 4])
                m1 = jnp.maximum(jnp.maximum(ms[0], ms[1]),
                                 jnp.maximum(ms[2], ms[3]))
                m2 = jnp.maximum(jnp.maximum(ms[4], ms[5]),
                                 jnp.maximum(ms[6], ms[7]))
                mrow = hmax(jnp.maximum(m1, m2))
                accs8 = [zv] * 8
                for c in range(_NFULL):
                    e = jnp.exp(buf[j, pl.ds(c * _L, _L)] - mrow)
                    accs8[c % 8] = accs8[c % 8] + e
                et = jnp.exp(buf[j, pl.ds(_TAIL, _L)] - mrow)
                accs8[62 % 8] = accs8[62 % 8] + jnp.where(tailmask, et, 0.0)
                s1 = (accs8[0] + accs8[1]) + (accs8[2] + accs8[3])
                s2 = (accs8[4] + accs8[5]) + (accs8[6] + accs8[7])
                srow = hsum(s1 + s2)
                tval = hsum((tvs[0] + tvs[1]) + (tvs[2] + tvs[3]))
                sel = lane == j2
                accm = jnp.where(sel, mrow, accm)
                accs = jnp.where(sel, srow, accs)
                acctv = jnp.where(sel, tval, acctv)
                return accm, accs, acctv

            zero = jnp.zeros((_L,), jnp.float32)
            accm, accs, acctv = lax.fori_loop(
                0, _L, row_body, (zero, zero, zero))
            off = base + b * _L
            mbuf[pl.ds(off, _L)] = accm
            sbuf[pl.ds(off, _L)] = accs
            tvbuf[pl.ds(off, _L)] = acctv
            avec = jnp.zeros((_L,), jnp.float32)
            for aoff in offsets:
                av_v = abuf[pl.ds(aoff, _L)]
                idx = jnp.clip(tvec16 - aoff, 0, _L - 1)
                hit = (tvec16 >= aoff) & (tvec16 < aoff + _L)
                avec = jnp.where(hit, _lgather(av_v, idx), avec)
            avbuf[pl.ds(off, _L)] = avec
            return _

        lax.fori_loop(0, _CH // _L, group_body, 0)

    start(0, xbuf0, sem0)

    def chunk_body(i, _):
        ch0 = i * 2

        @pl.when(ch0 + 1 < _NCHUNK)
        def _s1():
            start(ch0 + 1, xbuf1, sem1)

        wait(xbuf0, sem0)
        process(ch0, xbuf0)

        @pl.when(ch0 + 2 < _NCHUNK)
        def _s2():
            start(ch0 + 2, xbuf0, sem0)

        @pl.when(ch0 + 1 < _NCHUNK)
        def _p1():
            wait(xbuf1, sem1)
            process(ch0 + 1, xbuf1)

        return _

    lax.fori_loop(0, _NCHUNK // 2, chunk_body, 0)
    pltpu.sync_copy(mbuf, m_hbm.at[pl.ds(out0, _RPT)])
    pltpu.sync_copy(sbuf, s_hbm.at[pl.ds(out0, _RPT)])
    pltpu.sync_copy(tvbuf, tv_hbm.at[pl.ds(out0, _RPT)])
    pltpu.sync_copy(avbuf, av_hbm.at[pl.ds(out0, _RPT)])


_sc_call = pl.kernel(
    _sc_rows,
    out_type=[jax.ShapeDtypeStruct((_NSC,), jnp.float32)] * 4,
    mesh=plsc.VectorSubcoreMesh(core_axis_name="c", subcore_axis_name="s"),
    compiler_params=pltpu.CompilerParams(use_tc_tiling_on_sc=True),
    scratch_types=[
        pltpu.VMEM((_CH, _C), jnp.float32),
        pltpu.VMEM((_CH, _C), jnp.float32),
        pltpu.VMEM((_RPT,), jnp.int32),
        pltpu.VMEM((_C,), jnp.float32),
        pltpu.VMEM((_RPT,), jnp.float32),
        pltpu.VMEM((_RPT,), jnp.float32),
        pltpu.VMEM((_RPT,), jnp.float32),
        pltpu.VMEM((_RPT,), jnp.float32),
        pltpu.SemaphoreType.DMA,
        pltpu.SemaphoreType.DMA,
    ],
)


def _tc_main(x_ref, t_ref, a_ref, loss_ref):
    xb = x_ref[...]
    tg = t_ref[...]
    ar = a_ref[...]

    col = jax.lax.broadcasted_iota(jnp.int32, (_BT, _C), 1)
    onehot = (col == tg).astype(jnp.float32)
    tval = jnp.sum(xb * onehot, axis=1)
    aval = jnp.sum(ar * onehot, axis=1)

    m = jnp.max(xb, axis=1)
    s = jnp.sum(jnp.exp(xb - m[:, None]), axis=1)
    lp = tval - (m + jnp.log(s))
    p = jnp.exp(lp)
    omp = 1.0 - p
    loss_ref[...] = -aval * omp * omp * lp


def _f32_key(v):
    b = jax.lax.bitcast_convert_type(v, jnp.int32)
    return jnp.where(b >= 0, b, b ^ _IMAXP)


def _tc_fin(lb_ref, m_ref, s_ref, tv_ref, av_ref, out_ref):
    m = m_ref[...]
    s = s_ref[...]
    tv = tv_ref[...]
    av = av_ref[...]
    lp = tv - (m + jnp.log(s))
    p = jnp.exp(lp)
    omp = 1.0 - p
    loss_sc = -av * omp * omp * lp
    vals = jnp.concatenate([lb_ref[...], loss_sc])
    keys = _f32_key(vals)
    one = jnp.int32(1)

    def bit_step(b, tu):
        cand = tu | (one << (31 - b))
        cnt = jnp.sum((keys >= (cand ^ _IMIN)).astype(jnp.int32))
        return jnp.where(cnt >= _K, cand, tu)

    tu = jax.lax.fori_loop(0, 32, bit_step, jnp.int32(0))
    ti = tu ^ _IMIN
    tb = jnp.where(ti >= 0, ti, ti ^ _IMAXP)
    tau = jax.lax.bitcast_convert_type(tb, jnp.float32)
    gt = keys > ti
    cnt_gt = jnp.sum(gt.astype(jnp.int32))
    sum_gt = jnp.sum(jnp.where(gt, vals, 0.0))
    out_ref[0, 0] = (sum_gt + (_K - cnt_gt).astype(jnp.float32) * tau) / _K


def kernel(inputs, targets, alpha):
    a1 = alpha.reshape(-1)
    t2 = targets.reshape(_N, 1)
    ar = alpha.reshape(1, _C)
    m, s, tv, av = _sc_call(inputs, targets, a1)
    loss_bot = pl.pallas_call(
        _tc_main,
        grid=(_NBLK,),
        in_specs=[
            pl.BlockSpec((_BT, _C), lambda i: (i, 0)),
            pl.BlockSpec((_BT, 1), lambda i: (i, 0)),
            pl.BlockSpec((1, _C), lambda i: (0, 0)),
        ],
        out_specs=pl.BlockSpec((_BT,), lambda i: (i,)),
        out_shape=jax.ShapeDtypeStruct((_NTC,), jnp.float32),
    )(inputs, t2, ar)
    out = pl.pallas_call(
        _tc_fin,
        out_specs=pl.BlockSpec(memory_space=pltpu.SMEM),
        out_shape=jax.ShapeDtypeStruct((1, 1), jnp.float32),
    )(loss_bot, m, s, tv, av)
    return out[0, 0]


# speedup vs baseline: 2.8236x; 2.8236x over previous
import jax
import jax.numpy as jnp
from jax import lax
from jax.experimental import pallas as pl
from jax.experimental.pallas import tpu as pltpu

_N = 16384
_C = 1000
_K = int(_N * 0.2)
_BT = 2048
_NBLK = _N // _BT
_IMIN = -2**31
_IMAXP = 0x7FFFFFFF


def _f32_key(v):
    b = jax.lax.bitcast_convert_type(v, jnp.int32)
    return jnp.where(b >= 0, b, b ^ _IMAXP)


def _body(xt_ref, t_ref, a_ref, out_ref, loss_ref):
    i = pl.program_id(0)
    xb = xt_ref[...]
    tg = t_ref[...].reshape(1, _BT)
    al = a_ref[...]

    row = jax.lax.broadcasted_iota(jnp.int32, (_C, _BT), 0)
    oh = (row == tg).astype(jnp.float32)
    tval = jnp.sum(xb * oh, axis=0)
    aval = jnp.sum(al * oh, axis=0)

    m = jnp.max(xb, axis=0)
    s = jnp.sum(jnp.exp(xb - m[None, :]), axis=0)
    lp = tval - (m + jnp.log(s))
    p = jnp.exp(lp)
    omp = 1.0 - p
    loss_ref[pl.ds(i * _BT, _BT)] = -aval * omp * omp * lp

    @pl.when(i == _NBLK - 1)
    def _select():
        vals = loss_ref[...]
        keys = _f32_key(vals)
        one = jnp.int32(1)

        def bit_step(b, tu):
            cand = tu | (one << (31 - b))
            cnt = jnp.sum((keys >= (cand ^ _IMIN)).astype(jnp.int32))
            return jnp.where(cnt >= _K, cand, tu)

        tu = jax.lax.fori_loop(0, 32, bit_step, jnp.int32(0))
        ti = tu ^ _IMIN
        tb = jnp.where(ti >= 0, ti, ti ^ _IMAXP)
        tau = jax.lax.bitcast_convert_type(tb, jnp.float32)
        gt = keys > ti
        cnt_gt = jnp.sum(gt.astype(jnp.int32))
        sum_gt = jnp.sum(jnp.where(gt, vals, 0.0))
        out_ref[0, 0] = (sum_gt + (_K - cnt_gt).astype(jnp.float32) * tau) / _K


def kernel(inputs, targets, alpha):
    xt = inputs.T
    t3 = targets.reshape(_NBLK, 1, _BT)
    out = pl.pallas_call(
        _body,
        grid=(_NBLK,),
        in_specs=[
            pl.BlockSpec((_C, _BT), lambda i: (0, i)),
            pl.BlockSpec((1, 1, _BT), lambda i: (i, 0, 0)),
            pl.BlockSpec((_C, 1), lambda i: (0, 0)),
        ],
        out_specs=pl.BlockSpec(memory_space=pltpu.SMEM),
        out_shape=jax.ShapeDtypeStruct((1, 1), jnp.float32),
        scratch_shapes=[pltpu.VMEM((_N,), jnp.float32)],
    )(xt, t3, alpha)
    return out[0, 0]
